# baseline (speedup 1.0000x reference)
import functools

import jax
import jax.numpy as jnp
from jax import lax
from jax.experimental import pallas as pl
from jax.experimental.pallas import tpu as pltpu
from jax.experimental.pallas import tpu_sc as plsc

N = 10000
E = 320000
D = 128

NC = 2
NS = 16
NW = NC * NS
EPW = E // NW
CHUNK = 128
NCHUNK = 80
EPW_PAD = NCHUNK * CHUNK
N_PAD = 10240
TRASH_ROW = N
ROWS_PER_TILE = N_PAD // NS
WIN = 32
HWIN = WIN // 2
QG = 4

_MESH = plsc.VectorSubcoreMesh(core_axis_name="c", subcore_axis_name="s")


@functools.partial(
    pl.kernel,
    out_type=jax.ShapeDtypeStruct((NC, N_PAD, D), jnp.float32),
    mesh=_MESH,
    scratch_types=[
        pltpu.VMEM((WIN, CHUNK), jnp.int32),
        pltpu.VMEM((WIN, CHUNK), jnp.int32),
        pltpu.VMEM((WIN, CHUNK), jnp.float32),
        pltpu.VMEM((2 * CHUNK, D), jnp.float32),
        pltpu.VMEM_SHARED((N_PAD, D), jnp.float32),
        pltpu.SemaphoreType.DMA,
        pltpu.SemaphoreType.DMA,
        pltpu.SemaphoreType.DMA,
        pltpu.SemaphoreType.DMA,
    ],
)
def _sc_segment_sum(h_hbm, src_hbm, dst_hbm, w_hbm, zeros_hbm, out_hbm,
                    src_v, dst_v, w_v, ring_v, acc_sh, sg0, sg1, ss0, ss1):
    c = lax.axis_index("c")
    s = lax.axis_index("s")
    wid = s * NC + c

    pltpu.sync_copy(zeros_hbm,
                    acc_sh.at[pl.ds(s * ROWS_PER_TILE, ROWS_PER_TILE)])
    pltpu.sync_copy(src_hbm.at[wid, pl.ds(0, HWIN)],
                    src_v.at[pl.ds(0, HWIN)])
    pltpu.sync_copy(dst_hbm.at[wid, pl.ds(0, HWIN)],
                    dst_v.at[pl.ds(0, HWIN)])
    pltpu.sync_copy(w_hbm.at[wid, pl.ds(0, HWIN)],
                    w_v.at[pl.ds(0, HWIN)])
    plsc.subcore_barrier()

    def mul_chunk(off, wrow):
        def mul_body(g, carry):
            w16 = w_v[wrow, pl.ds(g * 16, 16)]
            e0 = off + g * 16
            for el in range(16):
                wv = jnp.full((16,), w16[el], dtype=jnp.float32)
                for t in range(D // 16):
                    sl = pl.ds(t * 16, 16)
                    ring_v[e0 + el, sl] = ring_v[e0 + el, sl] * wv
            return carry

        lax.fori_loop(0, CHUNK // 16, mul_body, 0)

    def pair_body(jg, carry):
        @pl.when(jnp.logical_and(lax.rem(jg, HWIN // 2) == 0,
                                 jg <= (NCHUNK - WIN) // 2))
        def _():
            jn = pl.multiple_of(jg * 2 + HWIN, HWIN)
            row = pl.multiple_of(lax.rem(jn, WIN), HWIN)
            pltpu.sync_copy(src_hbm.at[wid, pl.ds(jn, HWIN)],
                            src_v.at[pl.ds(row, HWIN)])
            pltpu.sync_copy(dst_hbm.at[wid, pl.ds(jn, HWIN)],
                            dst_v.at[pl.ds(row, HWIN)])
            pltpu.sync_copy(w_hbm.at[wid, pl.ds(jn, HWIN)],
                            w_v.at[pl.ds(row, HWIN)])

        r0 = lax.rem(jg * 2, WIN)
        r1 = r0 + 1
        QR = CHUNK // QG
        hg0 = [pltpu.async_copy(
                   h_hbm.at[src_v.at[r0, pl.ds(q * QR, QR)]],
                   ring_v.at[pl.ds(q * QR, QR)], sg0)
               for q in range(QG)]
        hg1 = [pltpu.async_copy(
                   h_hbm.at[src_v.at[r1, pl.ds(q * QR, QR)]],
                   ring_v.at[pl.ds(CHUNK + q * QR, QR)], sg1)
               for q in range(QG)]
        with jax.named_scope("g0wait"):
            for h in hg0:
                h.wait()
        with jax.named_scope("mul0"):
            mul_chunk(0, r0)
        hs0 = pltpu.async_copy(ring_v.at[pl.ds(0, CHUNK)],
                               acc_sh.at[dst_v.at[r0]], ss0, add=True)
        with jax.named_scope("g1wait"):
            for h in hg1:
                h.wait()
        with jax.named_scope("mul1"):
            mul_chunk(CHUNK, r1)
        hs1 = pltpu.async_copy(ring_v.at[pl.ds(CHUNK, CHUNK)],
                               acc_sh.at[dst_v.at[r1]], ss1, add=True)
        with jax.named_scope("stail"):
            hs0.wait()
            hs1.wait()
        return carry

    lax.fori_loop(0, NCHUNK // 2, pair_body, 0)
    plsc.subcore_barrier()

    pltpu.sync_copy(acc_sh.at[pl.ds(s * ROWS_PER_TILE, ROWS_PER_TILE)],
                    out_hbm.at[c, pl.ds(s * ROWS_PER_TILE, ROWS_PER_TILE)])


def _tc_body(relu, p_ref, h_ref, wr_ref, br_ref, wo_ref, a_ref, b_ref, o_ref):
    agg = p_ref[0, :N] + p_ref[1, :N]
    out = jnp.dot(agg, wr_ref[...], preferred_element_type=jnp.float32)
    out = out + jnp.dot(h_ref[...], wo_ref[...],
                        preferred_element_type=jnp.float32)
    out = out + br_ref[...]
    mean = jnp.mean(out, axis=-1, keepdims=True)
    cent = out - mean
    var = jnp.sum(cent * cent, axis=-1, keepdims=True) / (D - 1)
    y = a_ref[...] * cent / (jnp.sqrt(var) + 1e-6) + b_ref[...]
    if relu:
        y = jnp.maximum(y, 0.0)
    o_ref[...] = y


def _tc_stage(p, h, W_rel, b_rel, W_root, a, b, relu):
    return pl.pallas_call(
        functools.partial(_tc_body, relu),
        out_shape=jax.ShapeDtypeStruct((N, D), jnp.float32),
    )(p, h, W_rel, b_rel.reshape(1, D), W_root, a.reshape(1, D),
      b.reshape(1, D))


def kernel(x, edge_index, edge_weight,
           W_rel0, b_rel0, W_root0, a0, b0,
           W_rel1, b_rel1, W_root1, a1, b1,
           W_rel2, b_rel2, W_root2, af, bf):
    pad = EPW_PAD - EPW
    src = jnp.pad(edge_index[0].astype(jnp.int32).reshape(NW, EPW),
                  ((0, 0), (0, pad))).reshape(NW, NCHUNK, CHUNK)
    dst = jnp.pad(edge_index[1].astype(jnp.int32).reshape(NW, EPW),
                  ((0, 0), (0, pad)),
                  constant_values=TRASH_ROW).reshape(NW, NCHUNK, CHUNK)
    w = jnp.pad(edge_weight.reshape(NW, EPW),
                ((0, 0), (0, pad))).reshape(NW, NCHUNK, CHUNK)
    zeros = jnp.zeros((ROWS_PER_TILE, D), jnp.float32)

    h = x
    for (W_rel, b_rel, W_root, a, b, relu) in (
            (W_rel0, b_rel0, W_root0, a0, b0, True),
            (W_rel1, b_rel1, W_root1, a1, b1, True),
            (W_rel2, b_rel2, W_root2, af, bf, False)):
        p = _sc_segment_sum(h, src, dst, w, zeros)
        h = _tc_stage(p, h, W_rel, b_rel, W_root, a, b, relu)
    return h


# speedup vs baseline: 1.0094x; 1.0094x over previous
import functools

import jax
import jax.numpy as jnp
from jax import lax
from jax.experimental import pallas as pl
from jax.experimental.pallas import tpu as pltpu
from jax.experimental.pallas import tpu_sc as plsc

N = 10000
E = 320000
D = 128

NC = 2
NS = 16
NW = NC * NS
EPW = E // NW
CHUNK = 128
NCHUNK = 80
EPW_PAD = NCHUNK * CHUNK
N_PAD = 10240
TRASH_ROW = N
ROWS_PER_TILE = N_PAD // NS
WIN = 32
HWIN = WIN // 2
QG = 4

_MESH = plsc.VectorSubcoreMesh(core_axis_name="c", subcore_axis_name="s")


@functools.partial(
    pl.kernel,
    out_type=jax.ShapeDtypeStruct((NC, N_PAD, D), jnp.float32),
    mesh=_MESH,
    scratch_types=[
        pltpu.VMEM((WIN, CHUNK), jnp.int32),
        pltpu.VMEM((WIN, CHUNK), jnp.int32),
        pltpu.VMEM((WIN, CHUNK), jnp.float32),
        pltpu.VMEM((2 * CHUNK, D), jnp.float32),
        pltpu.VMEM_SHARED((N_PAD, D), jnp.float32),
        pltpu.SemaphoreType.DMA,
        pltpu.SemaphoreType.DMA,
        pltpu.SemaphoreType.DMA,
        pltpu.SemaphoreType.DMA,
    ],
)
def _sc_segment_sum(h_hbm, src_hbm, dst_hbm, w_hbm, zeros_hbm, out_hbm,
                    src_v, dst_v, w_v, ring_v, acc_sh, sg0, sg1, ss0, ss1):
    c = lax.axis_index("c")
    s = lax.axis_index("s")
    wid = s * NC + c

    pltpu.sync_copy(zeros_hbm,
                    acc_sh.at[pl.ds(s * ROWS_PER_TILE, ROWS_PER_TILE)])
    pltpu.sync_copy(src_hbm.at[wid, pl.ds(0, HWIN)],
                    src_v.at[pl.ds(0, HWIN)])
    pltpu.sync_copy(dst_hbm.at[wid, pl.ds(0, HWIN)],
                    dst_v.at[pl.ds(0, HWIN)])
    pltpu.sync_copy(w_hbm.at[wid, pl.ds(0, HWIN)],
                    w_v.at[pl.ds(0, HWIN)])
    plsc.subcore_barrier()

    def mul_chunk(off, wrow):
        def mul_body(g, carry):
            w16 = w_v[wrow, pl.ds(g * 16, 16)]
            e0 = off + g * 16
            for el in range(16):
                wv = jnp.full((16,), w16[el], dtype=jnp.float32)
                for t in range(D // 16):
                    sl = pl.ds(t * 16, 16)
                    ring_v[e0 + el, sl] = ring_v[e0 + el, sl] * wv
            return carry

        lax.fori_loop(0, CHUNK // 16, mul_body, 0)

    def pair_body(jg, carry):
        @pl.when(jnp.logical_and(lax.rem(jg, HWIN // 2) == 0,
                                 jg <= (NCHUNK - WIN) // 2))
        def _():
            jn = pl.multiple_of(jg * 2 + HWIN, HWIN)
            row = pl.multiple_of(lax.rem(jn, WIN), HWIN)
            pltpu.sync_copy(src_hbm.at[wid, pl.ds(jn, HWIN)],
                            src_v.at[pl.ds(row, HWIN)])
            pltpu.sync_copy(dst_hbm.at[wid, pl.ds(jn, HWIN)],
                            dst_v.at[pl.ds(row, HWIN)])
            pltpu.sync_copy(w_hbm.at[wid, pl.ds(jn, HWIN)],
                            w_v.at[pl.ds(row, HWIN)])

        r0 = lax.rem(jg * 2, WIN)
        r1 = r0 + 1
        hg0 = pltpu.async_copy(h_hbm.at[src_v.at[r0]],
                               ring_v.at[pl.ds(0, CHUNK)], sg0)
        with jax.named_scope("g0wait"):
            hg0.wait()
        hg1 = pltpu.async_copy(h_hbm.at[src_v.at[r1]],
                               ring_v.at[pl.ds(CHUNK, CHUNK)], sg1)
        with jax.named_scope("mul0"):
            mul_chunk(0, r0)
        hs0 = pltpu.async_copy(ring_v.at[pl.ds(0, CHUNK)],
                               acc_sh.at[dst_v.at[r0]], ss0, add=True)
        with jax.named_scope("g1wait"):
            hg1.wait()
        with jax.named_scope("mul1"):
            mul_chunk(CHUNK, r1)
        hs1 = pltpu.async_copy(ring_v.at[pl.ds(CHUNK, CHUNK)],
                               acc_sh.at[dst_v.at[r1]], ss1, add=True)
        with jax.named_scope("stail"):
            hs0.wait()
            hs1.wait()
        return carry

    lax.fori_loop(0, NCHUNK // 2, pair_body, 0)
    plsc.subcore_barrier()

    pltpu.sync_copy(acc_sh.at[pl.ds(s * ROWS_PER_TILE, ROWS_PER_TILE)],
                    out_hbm.at[c, pl.ds(s * ROWS_PER_TILE, ROWS_PER_TILE)])


def _tc_body(relu, p_ref, h_ref, wr_ref, br_ref, wo_ref, a_ref, b_ref, o_ref):
    agg = p_ref[0, :N] + p_ref[1, :N]
    out = jnp.dot(agg, wr_ref[...], preferred_element_type=jnp.float32)
    out = out + jnp.dot(h_ref[...], wo_ref[...],
                        preferred_element_type=jnp.float32)
    out = out + br_ref[...]
    mean = jnp.mean(out, axis=-1, keepdims=True)
    cent = out - mean
    var = jnp.sum(cent * cent, axis=-1, keepdims=True) / (D - 1)
    y = a_ref[...] * cent / (jnp.sqrt(var) + 1e-6) + b_ref[...]
    if relu:
        y = jnp.maximum(y, 0.0)
    o_ref[...] = y


def _tc_stage(p, h, W_rel, b_rel, W_root, a, b, relu):
    return pl.pallas_call(
        functools.partial(_tc_body, relu),
        out_shape=jax.ShapeDtypeStruct((N, D), jnp.float32),
    )(p, h, W_rel, b_rel.reshape(1, D), W_root, a.reshape(1, D),
      b.reshape(1, D))


def kernel(x, edge_index, edge_weight,
           W_rel0, b_rel0, W_root0, a0, b0,
           W_rel1, b_rel1, W_root1, a1, b1,
           W_rel2, b_rel2, W_root2, af, bf):
    pad = EPW_PAD - EPW
    src = jnp.pad(edge_index[0].astype(jnp.int32).reshape(NW, EPW),
                  ((0, 0), (0, pad))).reshape(NW, NCHUNK, CHUNK)
    dst = jnp.pad(edge_index[1].astype(jnp.int32).reshape(NW, EPW),
                  ((0, 0), (0, pad)),
                  constant_values=TRASH_ROW).reshape(NW, NCHUNK, CHUNK)
    w = jnp.pad(edge_weight.reshape(NW, EPW),
                ((0, 0), (0, pad))).reshape(NW, NCHUNK, CHUNK)
    zeros = jnp.zeros((ROWS_PER_TILE, D), jnp.float32)

    h = x
    for (W_rel, b_rel, W_root, a, b, relu) in (
            (W_rel0, b_rel0, W_root0, a0, b0, True),
            (W_rel1, b_rel1, W_root1, a1, b1, True),
            (W_rel2, b_rel2, W_root2, af, bf, False)):
        p = _sc_segment_sum(h, src, dst, w, zeros)
        h = _tc_stage(p, h, W_rel, b_rel, W_root, a, b, relu)
    return h
